# Initial kernel scaffold; baseline (speedup 1.0000x reference)
import jax
import jax.numpy as jnp
from jax.experimental import pallas as pl


def kernel(z, edge_index, edge_attr, emb, W1, b1, W2, b2):
    raise NotImplementedError("write your pallas kernel here")


# speedup vs baseline: 3.5409x; 3.5409x over previous
import functools

import jax
import jax.numpy as jnp
from jax import lax
from jax.experimental import pallas as pl
from jax.experimental.pallas import tpu as pltpu
from jax.experimental.pallas import tpu_sc as plsc

N = 10000
E = 320000
D = 128

NC = 2
NS = 16
NW = NC * NS

NPAD = 10240
EMB_PER_W = NPAD // NW
EMB_CH = 80
EMB_NCH = EMB_PER_W // EMB_CH

EPT = E // NW
CH = 80
NCHUNK = EPT // CH
WB = 624
REM = N - NS * WB

_mesh = plsc.VectorSubcoreMesh(core_axis_name="c", subcore_axis_name="s")


@functools.partial(
    pl.kernel,
    out_type=jax.ShapeDtypeStruct((NPAD, D), jnp.float32),
    mesh=_mesh,
    scratch_types=[
        pltpu.VMEM((EMB_CH,), jnp.int32),
        pltpu.VMEM((EMB_CH, D), jnp.float32),
        pltpu.SemaphoreType.DMA,
    ],
)
def _embed_sc(emb_hbm, z_hbm, out_hbm, zi_v, row_v, sem):
    c = lax.axis_index("c")
    s = lax.axis_index("s")
    w = s * NC + c
    base = w * EMB_PER_W
    for k in range(EMB_NCH):
        pltpu.sync_copy(z_hbm.at[pl.ds(base + k * EMB_CH, EMB_CH)], zi_v)
        pltpu.async_copy(emb_hbm.at[zi_v], row_v, sem).wait()
        pltpu.sync_copy(row_v, out_hbm.at[pl.ds(base + k * EMB_CH, EMB_CH)])


@functools.partial(
    pl.kernel,
    out_type=jax.ShapeDtypeStruct((NC, N, D), jnp.float32),
    mesh=_mesh,
    scratch_types=[
        pltpu.VMEM((CH,), jnp.int32),
        pltpu.VMEM((CH,), jnp.int32),
        pltpu.VMEM((CH, D), jnp.float32),
        pltpu.VMEM((CH, D), jnp.float32),
        pltpu.VMEM_SHARED((N, D), jnp.float32),
        pltpu.SemaphoreType.DMA,
    ],
)
def _msg_sc(x_hbm, src_hbm, dst_hbm, ea_hbm, out_hbm, src_v, dst_v, xr_v, ms_v,
            agg_sh, sem):
    c = lax.axis_index("c")
    s = lax.axis_index("s")
    w = s * NC + c

    def _zero_row(r, _):
        for g in range(D // 16):
            ms_v[r, pl.ds(g * 16, 16)] = jnp.zeros((16,), jnp.float32)
        return _
    lax.fori_loop(0, CH, _zero_row, 0)
    r0 = s * WB
    for k in range(WB // CH):
        pltpu.sync_copy(ms_v, agg_sh.at[pl.ds(r0 + k * CH, CH)])
    pltpu.sync_copy(ms_v.at[pl.ds(0, WB - (WB // CH) * CH)],
                    agg_sh.at[pl.ds(r0 + (WB // CH) * CH, WB - (WB // CH) * CH)])

    @pl.when(s == NS - 1)
    def _zero_tail():
        pltpu.sync_copy(ms_v.at[pl.ds(0, REM)],
                        agg_sh.at[pl.ds(NS * WB, REM)])
    plsc.subcore_barrier()

    ebase = w * EPT

    def _chunk(j, _):
        off = ebase + j * CH
        pltpu.sync_copy(src_hbm.at[pl.ds(off, CH)], src_v)
        pltpu.sync_copy(dst_hbm.at[pl.ds(off, CH)], dst_v)
        gather = pltpu.async_copy(x_hbm.at[src_v], xr_v, sem)
        pltpu.sync_copy(ea_hbm.at[pl.ds(off, CH)], ms_v)
        gather.wait()

        def _row(r, _2):
            for g in range(D // 16):
                sl = pl.ds(g * 16, 16)
                ms_v[r, sl] = jnp.maximum(ms_v[r, sl] + xr_v[r, sl], 0.0)
            return _2
        lax.fori_loop(0, CH, _row, 0)
        pltpu.sync_copy(ms_v, agg_sh.at[dst_v], add=True)
        return _
    lax.fori_loop(0, NCHUNK, _chunk, 0)

    plsc.subcore_barrier()
    pltpu.sync_copy(agg_sh.at[pl.ds(r0, WB)], out_hbm.at[c].at[pl.ds(r0, WB)])

    @pl.when(s == NS - 1)
    def _write_tail():
        pltpu.sync_copy(agg_sh.at[pl.ds(NS * WB, REM)],
                        out_hbm.at[c].at[pl.ds(NS * WB, REM)])


MLP_B = 1000


def _mlp_body(p0, p1, x, w1, b1, w2, b2, o, *, out_relu):
    sagg = p0[...] + p1[...] + x[...]
    h = jnp.maximum(
        jnp.dot(sagg, w1[...], preferred_element_type=jnp.float32) + b1[...], 0.0)
    y = jnp.dot(h, w2[...], preferred_element_type=jnp.float32) + b2[...]
    if out_relu:
        y = jnp.maximum(y, 0.0)
    o[...] = y + x[...]


def _mlp(partials, x, w1, b1, w2, b2, out_relu):
    row_spec = pl.BlockSpec((MLP_B, D), lambda i: (i, 0))
    full_spec = pl.BlockSpec((D, D), lambda i: (0, 0))
    bias_spec = pl.BlockSpec((1, D), lambda i: (0, 0))
    return pl.pallas_call(
        functools.partial(_mlp_body, out_relu=out_relu),
        grid=(N // MLP_B,),
        in_specs=[row_spec, row_spec, row_spec, full_spec, bias_spec,
                  full_spec, bias_spec],
        out_specs=row_spec,
        out_shape=jax.ShapeDtypeStruct((N, D), jnp.float32),
    )(partials[0], partials[1], x, w1, b1.reshape(1, D), w2, b2.reshape(1, D))


def kernel(z, edge_index, edge_attr, emb, W1, b1, W2, b2):
    z = z.astype(jnp.int32)
    z_pad = jnp.concatenate([z, jnp.zeros((NPAD - N,), jnp.int32)])
    x = _embed_sc(emb, z_pad)[:N]

    src = edge_index[0]
    dst = edge_index[1]

    for i in range(3):
        partials = _msg_sc(x, src, dst, edge_attr)
        x = _mlp(partials, x, W1[i], b1[i], W2[i], b2[i], out_relu=(i < 2))
    return x
